# Initial kernel scaffold; baseline (speedup 1.0000x reference)
import jax
import jax.numpy as jnp
from jax.experimental import pallas as pl


def kernel(x, edge_index, batch, eps1, W1a, b1a, W2a, b2a, eps2, W1b, b1b, W2b, b2b, Wp1, bp1, Wp2, bp2, Wo, bo):
    raise NotImplementedError("write your pallas kernel here")


# speedup vs baseline: 8.5988x; 8.5988x over previous
import functools

import jax
import jax.numpy as jnp
from jax import lax
from jax.experimental import pallas as pl
from jax.experimental.pallas import tpu as pltpu
from jax.experimental.pallas import tpu_sc as plsc

N = 100000
E = 1600000
G = 64

NC = 2
NS = 16
LB = 128
SB = 1024
PT = 50176
E_PAD = NC * NS * PT
NBLK = PT // SB
ROWS_PER_TILE = 6256
ACC_ROWS = NS * ROWS_PER_TILE

BN = 5000
NGRID = N // BN


def _ceil_chunks(total, step):
    out, o = [], 0
    while o < total:
        out.append((o, min(step, total - o)))
        o += step
    return out


def _make_agg(num_passes):
    mesh = plsc.VectorSubcoreMesh(
        core_axis_name="c", subcore_axis_name="s", num_cores=NC, num_subcores=NS)

    @functools.partial(
        pl.kernel,
        out_type=jax.ShapeDtypeStruct((NC, ACC_ROWS, 16 * num_passes),
                                      jnp.float32),
        mesh=mesh,
        scratch_types=[
            pltpu.VMEM((8, LB), jnp.int32),
            pltpu.VMEM((8, LB), jnp.int32),
            pltpu.VMEM((SB, 16), jnp.float32),
            pltpu.VMEM_SHARED((ACC_ROWS, 16), jnp.float32),
            pltpu.SemaphoreType.DMA,
        ],
        compiler_params=pltpu.CompilerParams(use_tc_tiling_on_sc=False),
    )
    def agg(src_hbm, dst_hbm, tbl, out, src_idx, dst_idx, rows, acc, sem):
        cid = lax.axis_index("c")
        sid = lax.axis_index("s")
        roff = (cid * (NS * PT) + sid * PT) // LB
        base = pl.multiple_of(sid * ROWS_PER_TILE, 8)

        for p in range(num_passes):
            def zb(r, carry):
                rows[r, :] = jnp.zeros((16,), jnp.float32)
                return carry
            lax.fori_loop(0, SB, zb, 0)
            for (o, sz) in _ceil_chunks(ROWS_PER_TILE, SB):
                pltpu.sync_copy(rows.at[pl.ds(0, sz), :],
                                acc.at[pl.ds(pl.multiple_of(base + o, 8), sz), :])
            plsc.subcore_barrier()

            def body(b, carry):
                r0 = pl.multiple_of(roff + b * 8, 8)
                pltpu.sync_copy(src_hbm.at[pl.ds(r0, 8), :], src_idx)
                pltpu.sync_copy(dst_hbm.at[pl.ds(r0, 8), :], dst_idx)
                if num_passes > 1:
                    for j in range(8):
                        for k in range(LB // 16):
                            sl = pl.ds(k * 16, 16)
                            src_idx[j, sl] = src_idx[j, sl] * num_passes + p
                cps = [pltpu.async_copy(tbl.at[src_idx.at[j]],
                                        rows.at[pl.ds(j * LB, LB), :], sem)
                       for j in range(8)]
                for c in cps:
                    c.wait()
                for j in range(8):
                    pltpu.sync_copy(rows.at[pl.ds(j * LB, LB), :],
                                    acc.at[dst_idx.at[j]], add=True)
                return carry
            lax.fori_loop(0, NBLK, body, 0)
            plsc.subcore_barrier()

            for (o, sz) in _ceil_chunks(ROWS_PER_TILE, SB):
                o8 = pl.multiple_of(base + o, 8)
                pltpu.sync_copy(acc.at[pl.ds(o8, sz), :],
                                out.at[cid, pl.ds(o8, sz), pl.ds(16 * p, 16)])
            plsc.subcore_barrier()

    return agg


def _mlp1_body(eps_ref, xp_ref, agg_ref, w1_ref, b1_ref, w2_ref, b2_ref, out):
    x = xp_ref[...]
    agg = agg_ref[0] + agg_ref[1]
    h = (1.0 + eps_ref[0, 0]) * x + agg
    h = jnp.maximum(jnp.dot(h, w1_ref[...],
                            preferred_element_type=jnp.float32) + b1_ref[...], 0.0)
    h = jnp.dot(h, w2_ref[...], preferred_element_type=jnp.float32) + b2_ref[...]
    out[...] = jnp.maximum(h, 0.0)


def _mlp2_body(eps_ref, h1_ref, agg_ref, batch_ref,
               w1_ref, b1_ref, w2_ref, b2_ref, wp1_ref, bp1_ref,
               wp2_ref, bp2_ref, wo_ref, bo_ref,
               pres_out, ord_out, pool_acc):
    i = pl.program_id(0)
    h1 = h1_ref[...]
    agg = agg_ref[0] + agg_ref[1]
    h = (1.0 + eps_ref[0, 0]) * h1 + agg
    h = jnp.maximum(jnp.dot(h, w1_ref[...],
                            preferred_element_type=jnp.float32) + b1_ref[...], 0.0)
    h = jnp.dot(h, w2_ref[...], preferred_element_type=jnp.float32) + b2_ref[...]
    h2 = jnp.maximum(h, 0.0)

    bids = batch_ref[0, 0, :]
    oneh = (lax.broadcasted_iota(jnp.int32, (G, 1), 0)
            == bids[None, :]).astype(jnp.float32)
    part = jax.lax.dot_general(oneh, h2, (((1,), (0,)), ((), ())),
                               preferred_element_type=jnp.float32)

    @pl.when(i == 0)
    def _():
        pool_acc[...] = jnp.zeros_like(pool_acc)

    pool_acc[...] += part

    @pl.when(i == pl.num_programs(0) - 1)
    def _():
        pool = pool_acc[...]
        pr = jnp.maximum(jnp.dot(pool, wp1_ref[...],
                                 preferred_element_type=jnp.float32) + bp1_ref[...], 0.0)
        pres_out[...] = jax.nn.sigmoid(
            jnp.dot(pr, wp2_ref[...], preferred_element_type=jnp.float32) + bp2_ref[...])
        ord_out[...] = jnp.dot(pool, wo_ref[...],
                               preferred_element_type=jnp.float32) + bo_ref[...]


def _full(shape):
    return pl.BlockSpec(shape, lambda i: tuple(0 for _ in shape))


def kernel(x, edge_index, batch, eps1, W1a, b1a, W2a, b2a,
           eps2, W1b, b1b, W2b, b2b, Wp1, bp1, Wp2, bp2, Wo, bo):
    f32 = jnp.float32
    pad_e = E_PAD - E
    src2d = jnp.concatenate(
        [edge_index[0], jnp.zeros((pad_e,), jnp.int32)]).reshape(-1, LB)
    dst2d = jnp.concatenate(
        [edge_index[1], jnp.full((pad_e,), N, jnp.int32)]).reshape(-1, LB)
    x_pad = jnp.pad(x, ((0, 0), (0, 16 - x.shape[1])))
    W1a_p = jnp.pad(W1a, ((0, 16 - W1a.shape[0]), (0, 0)))
    batch3 = batch.reshape(NGRID, 1, BN)

    agg1 = _make_agg(1)(src2d, dst2d, x_pad)

    eps1s = eps1.reshape(1, 1)
    eps2s = eps2.reshape(1, 1)

    smem_spec = pl.BlockSpec(memory_space=pltpu.SMEM)

    h1 = pl.pallas_call(
        _mlp1_body,
        grid=(NGRID,),
        in_specs=[smem_spec,
                  pl.BlockSpec((BN, 16), lambda i: (i, 0)),
                  pl.BlockSpec((NC, BN, 16), lambda i: (0, i, 0)),
                  _full((16, 64)), _full((1, 64)), _full((64, 64)), _full((1, 64))],
        out_specs=pl.BlockSpec((BN, 64), lambda i: (i, 0)),
        out_shape=jax.ShapeDtypeStruct((N, 64), f32),
    )(eps1s, x_pad, agg1, W1a_p, b1a.reshape(1, 64), W2a, b2a.reshape(1, 64))

    h1_4 = h1.reshape(4 * N, 16)
    agg2 = _make_agg(4)(src2d, dst2d, h1_4)

    pres, ord24 = pl.pallas_call(
        _mlp2_body,
        grid=(NGRID,),
        in_specs=[smem_spec,
                  pl.BlockSpec((BN, 64), lambda i: (i, 0)),
                  pl.BlockSpec((NC, BN, 64), lambda i: (0, i, 0)),
                  pl.BlockSpec((1, 1, BN), lambda i: (i, 0, 0)),
                  _full((64, 64)), _full((1, 64)), _full((64, 64)), _full((1, 64)),
                  _full((64, 32)), _full((1, 32)), _full((32, 8)), _full((1, 8)),
                  _full((64, 24)), _full((1, 24))],
        out_specs=[_full((G, 8)), _full((G, 24))],
        out_shape=[jax.ShapeDtypeStruct((G, 8), f32),
                   jax.ShapeDtypeStruct((G, 24), f32)],
        scratch_shapes=[pltpu.VMEM((G, 64), f32)],
    )(eps2s, h1, agg2, batch3,
      W1b, b1b.reshape(1, 64), W2b, b2b.reshape(1, 64),
      Wp1, bp1.reshape(1, 32), Wp2, bp2.reshape(1, 8),
      Wo, bo.reshape(1, 24))

    return pres, ord24.reshape(G, 3, 8)


# speedup vs baseline: 9.8506x; 1.1456x over previous
import functools

import jax
import jax.numpy as jnp
from jax import lax
from jax.experimental import pallas as pl
from jax.experimental.pallas import tpu as pltpu
from jax.experimental.pallas import tpu_sc as plsc

N = 100000
E = 1600000
G = 64

NC = 2
NS = 16
LB = 128
SB = 1024
PT = 50176
E_PAD = NC * NS * PT
NBLK = PT // SB
ROWS_PER_TILE = 6256
ACC_ROWS = NS * ROWS_PER_TILE

BN = 5000
NGRID = N // BN


def _ceil_chunks(total, step):
    out, o = [], 0
    while o < total:
        out.append((o, min(step, total - o)))
        o += step
    return out


def _make_agg(num_passes):
    mesh = plsc.VectorSubcoreMesh(
        core_axis_name="c", subcore_axis_name="s", num_cores=NC, num_subcores=NS)

    @functools.partial(
        pl.kernel,
        out_type=jax.ShapeDtypeStruct((NC, ACC_ROWS, 16 * num_passes),
                                      jnp.float32),
        mesh=mesh,
        scratch_types=[
            pltpu.VMEM((8, LB), jnp.int32),
            pltpu.VMEM((8, LB), jnp.int32),
            pltpu.VMEM((SB, 16), jnp.float32),
            pltpu.VMEM_SHARED((ACC_ROWS, 16), jnp.float32),
            pltpu.SemaphoreType.DMA,
            pltpu.SemaphoreType.DMA,
        ],
        compiler_params=pltpu.CompilerParams(use_tc_tiling_on_sc=False),
    )
    def agg(src_hbm, dst_hbm, tbl, out, src_idx, dst_idx, rows, acc, sem, ssem):
        cid = lax.axis_index("c")
        sid = lax.axis_index("s")
        roff = (cid * (NS * PT) + sid * PT) // LB
        base = pl.multiple_of(sid * ROWS_PER_TILE, 8)

        for p in range(num_passes):
            def zb(r, carry):
                rows[r, :] = jnp.zeros((16,), jnp.float32)
                return carry
            lax.fori_loop(0, SB, zb, 0)
            for (o, sz) in _ceil_chunks(ROWS_PER_TILE, SB):
                pltpu.sync_copy(rows.at[pl.ds(0, sz), :],
                                acc.at[pl.ds(pl.multiple_of(base + o, 8), sz), :])
            plsc.subcore_barrier()

            def body(b, carry):
                r0 = pl.multiple_of(roff + b * 8, 8)
                pltpu.sync_copy(src_hbm.at[pl.ds(r0, 8), :], src_idx)
                pltpu.sync_copy(dst_hbm.at[pl.ds(r0, 8), :], dst_idx)
                if num_passes > 1:
                    for j in range(8):
                        for k in range(LB // 16):
                            sl = pl.ds(k * 16, 16)
                            src_idx[j, sl] = src_idx[j, sl] * num_passes + p
                cps = [pltpu.async_copy(tbl.at[src_idx.at[j]],
                                        rows.at[pl.ds(j * LB, LB), :], sem)
                       for j in range(8)]
                scs = []
                for j in range(8):
                    cps[j].wait()
                    scs.append(pltpu.async_copy(rows.at[pl.ds(j * LB, LB), :],
                                                acc.at[dst_idx.at[j]], ssem,
                                                add=True))
                for s in scs:
                    s.wait()
                return carry
            lax.fori_loop(0, NBLK, body, 0)
            plsc.subcore_barrier()

            for (o, sz) in _ceil_chunks(ROWS_PER_TILE, SB):
                o8 = pl.multiple_of(base + o, 8)
                pltpu.sync_copy(acc.at[pl.ds(o8, sz), :],
                                out.at[cid, pl.ds(o8, sz), pl.ds(16 * p, 16)])
            plsc.subcore_barrier()

    return agg


def _mlp1_body(eps_ref, xp_ref, agg_ref, w1_ref, b1_ref, w2_ref, b2_ref, out):
    x = xp_ref[...]
    agg = agg_ref[0] + agg_ref[1]
    h = (1.0 + eps_ref[0, 0]) * x + agg
    h = jnp.maximum(jnp.dot(h, w1_ref[...],
                            preferred_element_type=jnp.float32) + b1_ref[...], 0.0)
    h = jnp.dot(h, w2_ref[...], preferred_element_type=jnp.float32) + b2_ref[...]
    out[...] = jnp.maximum(h, 0.0)


def _mlp2_body(eps_ref, h1_ref, agg_ref, batch_ref,
               w1_ref, b1_ref, w2_ref, b2_ref, wp1_ref, bp1_ref,
               wp2_ref, bp2_ref, wo_ref, bo_ref,
               pres_out, ord_out, pool_acc):
    i = pl.program_id(0)
    h1 = h1_ref[...]
    agg = agg_ref[0] + agg_ref[1]
    h = (1.0 + eps_ref[0, 0]) * h1 + agg
    h = jnp.maximum(jnp.dot(h, w1_ref[...],
                            preferred_element_type=jnp.float32) + b1_ref[...], 0.0)
    h = jnp.dot(h, w2_ref[...], preferred_element_type=jnp.float32) + b2_ref[...]
    h2 = jnp.maximum(h, 0.0)

    bids = batch_ref[0, 0, :]
    oneh = (lax.broadcasted_iota(jnp.int32, (G, 1), 0)
            == bids[None, :]).astype(jnp.float32)
    part = jax.lax.dot_general(oneh, h2, (((1,), (0,)), ((), ())),
                               preferred_element_type=jnp.float32)

    @pl.when(i == 0)
    def _():
        pool_acc[...] = jnp.zeros_like(pool_acc)

    pool_acc[...] += part

    @pl.when(i == pl.num_programs(0) - 1)
    def _():
        pool = pool_acc[...]
        pr = jnp.maximum(jnp.dot(pool, wp1_ref[...],
                                 preferred_element_type=jnp.float32) + bp1_ref[...], 0.0)
        pres_out[...] = jax.nn.sigmoid(
            jnp.dot(pr, wp2_ref[...], preferred_element_type=jnp.float32) + bp2_ref[...])
        ord_out[...] = jnp.dot(pool, wo_ref[...],
                               preferred_element_type=jnp.float32) + bo_ref[...]


def _full(shape):
    return pl.BlockSpec(shape, lambda i: tuple(0 for _ in shape))


def kernel(x, edge_index, batch, eps1, W1a, b1a, W2a, b2a,
           eps2, W1b, b1b, W2b, b2b, Wp1, bp1, Wp2, bp2, Wo, bo):
    f32 = jnp.float32
    pad_e = E_PAD - E
    src2d = jnp.concatenate(
        [edge_index[0], jnp.zeros((pad_e,), jnp.int32)]).reshape(-1, LB)
    dst2d = jnp.concatenate(
        [edge_index[1], jnp.full((pad_e,), N, jnp.int32)]).reshape(-1, LB)
    x_pad = jnp.pad(x, ((0, 0), (0, 16 - x.shape[1])))
    W1a_p = jnp.pad(W1a, ((0, 16 - W1a.shape[0]), (0, 0)))
    batch3 = batch.reshape(NGRID, 1, BN)

    agg1 = _make_agg(1)(src2d, dst2d, x_pad)

    eps1s = eps1.reshape(1, 1)
    eps2s = eps2.reshape(1, 1)

    smem_spec = pl.BlockSpec(memory_space=pltpu.SMEM)

    h1 = pl.pallas_call(
        _mlp1_body,
        grid=(NGRID,),
        in_specs=[smem_spec,
                  pl.BlockSpec((BN, 16), lambda i: (i, 0)),
                  pl.BlockSpec((NC, BN, 16), lambda i: (0, i, 0)),
                  _full((16, 64)), _full((1, 64)), _full((64, 64)), _full((1, 64))],
        out_specs=pl.BlockSpec((BN, 64), lambda i: (i, 0)),
        out_shape=jax.ShapeDtypeStruct((N, 64), f32),
    )(eps1s, x_pad, agg1, W1a_p, b1a.reshape(1, 64), W2a, b2a.reshape(1, 64))

    h1_4 = h1.reshape(4 * N, 16)
    agg2 = _make_agg(4)(src2d, dst2d, h1_4)

    pres, ord24 = pl.pallas_call(
        _mlp2_body,
        grid=(NGRID,),
        in_specs=[smem_spec,
                  pl.BlockSpec((BN, 64), lambda i: (i, 0)),
                  pl.BlockSpec((NC, BN, 64), lambda i: (0, i, 0)),
                  pl.BlockSpec((1, 1, BN), lambda i: (i, 0, 0)),
                  _full((64, 64)), _full((1, 64)), _full((64, 64)), _full((1, 64)),
                  _full((64, 32)), _full((1, 32)), _full((32, 8)), _full((1, 8)),
                  _full((64, 24)), _full((1, 24))],
        out_specs=[_full((G, 8)), _full((G, 24))],
        out_shape=[jax.ShapeDtypeStruct((G, 8), f32),
                   jax.ShapeDtypeStruct((G, 24), f32)],
        scratch_shapes=[pltpu.VMEM((G, 64), f32)],
    )(eps2s, h1, agg2, batch3,
      W1b, b1b.reshape(1, 64), W2b, b2b.reshape(1, 64),
      Wp1, bp1.reshape(1, 32), Wp2, bp2.reshape(1, 8),
      Wo, bo.reshape(1, 24))

    return pres, ord24.reshape(G, 3, 8)


# speedup vs baseline: 11.2288x; 1.1399x over previous
import functools

import jax
import jax.numpy as jnp
from jax import lax
from jax.experimental import pallas as pl
from jax.experimental.pallas import tpu as pltpu
from jax.experimental.pallas import tpu_sc as plsc

N = 100000
E = 1600000
G = 64

NC = 2
NS = 16
LB = 128
SB = 1024
PT = 50176
E_PAD = NC * NS * PT
NBLK = PT // SB
NSUP = 7
ROWS_PER_TILE = 6256
ACC_ROWS = NS * ROWS_PER_TILE

BN = 5000
NGRID = N // BN


def _ceil_chunks(total, step):
    out, o = [], 0
    while o < total:
        out.append((o, min(step, total - o)))
        o += step
    return out


def _make_agg(num_passes):
    mesh = plsc.VectorSubcoreMesh(
        core_axis_name="c", subcore_axis_name="s", num_cores=NC, num_subcores=NS)

    @functools.partial(
        pl.kernel,
        out_type=jax.ShapeDtypeStruct((NC, ACC_ROWS, 16 * num_passes),
                                      jnp.float32),
        mesh=mesh,
        scratch_types=[
            pltpu.VMEM((56, LB), jnp.int32),
            pltpu.VMEM((56, LB), jnp.int32),
            pltpu.VMEM((SB, 16), jnp.float32),
            pltpu.VMEM_SHARED((ACC_ROWS, 16), jnp.float32),
            pltpu.SemaphoreType.DMA,
            pltpu.SemaphoreType.DMA,
        ],
        compiler_params=pltpu.CompilerParams(use_tc_tiling_on_sc=False),
    )
    def agg(src_hbm, dst_hbm, tbl, out, src_idx, dst_idx, rows, acc, sem, ssem):
        cid = lax.axis_index("c")
        sid = lax.axis_index("s")
        roff = (cid * (NS * PT) + sid * PT) // LB
        base = pl.multiple_of(sid * ROWS_PER_TILE, 8)

        for p in range(num_passes):
            def zb(r, carry):
                rows[r, :] = jnp.zeros((16,), jnp.float32)
                return carry
            lax.fori_loop(0, SB, zb, 0)
            for (o, sz) in _ceil_chunks(ROWS_PER_TILE, SB):
                pltpu.sync_copy(rows.at[pl.ds(0, sz), :],
                                acc.at[pl.ds(pl.multiple_of(base + o, 8), sz), :])
            plsc.subcore_barrier()

            def body(sc, carry):
                r0 = pl.multiple_of(roff + sc * (8 * NSUP), 8)
                pltpu.sync_copy(src_hbm.at[pl.ds(r0, 8 * NSUP), :], src_idx)
                pltpu.sync_copy(dst_hbm.at[pl.ds(r0, 8 * NSUP), :], dst_idx)
                if num_passes > 1:
                    def scale(j, c2):
                        for k in range(LB // 16):
                            sl = pl.ds(k * 16, 16)
                            src_idx[j, sl] = src_idx[j, sl] * num_passes + p
                        return c2
                    lax.fori_loop(0, 8 * NSUP, scale, 0)
                for b in range(NSUP):
                    cps = [pltpu.async_copy(tbl.at[src_idx.at[b * 8 + j]],
                                            rows.at[pl.ds(j * LB, LB), :], sem)
                           for j in range(8)]
                    scs = []
                    for j in range(8):
                        cps[j].wait()
                        scs.append(
                            pltpu.async_copy(rows.at[pl.ds(j * LB, LB), :],
                                             acc.at[dst_idx.at[b * 8 + j]],
                                             ssem, add=True))
                    for s in scs:
                        s.wait()
                return carry
            lax.fori_loop(0, NBLK // NSUP, body, 0)
            plsc.subcore_barrier()

            for (o, sz) in _ceil_chunks(ROWS_PER_TILE, SB):
                o8 = pl.multiple_of(base + o, 8)
                pltpu.sync_copy(acc.at[pl.ds(o8, sz), :],
                                out.at[cid, pl.ds(o8, sz), pl.ds(16 * p, 16)])
            plsc.subcore_barrier()

    return agg


def _mlp1_body(eps_ref, xp_ref, agg_ref, w1_ref, b1_ref, w2_ref, b2_ref, out):
    x = xp_ref[...]
    agg = agg_ref[0] + agg_ref[1]
    h = (1.0 + eps_ref[0, 0]) * x + agg
    h = jnp.maximum(jnp.dot(h, w1_ref[...],
                            preferred_element_type=jnp.float32) + b1_ref[...], 0.0)
    h = jnp.dot(h, w2_ref[...], preferred_element_type=jnp.float32) + b2_ref[...]
    out[...] = jnp.maximum(h, 0.0)


def _mlp2_body(eps_ref, h1_ref, agg_ref, batch_ref,
               w1_ref, b1_ref, w2_ref, b2_ref, wp1_ref, bp1_ref,
               wp2_ref, bp2_ref, wo_ref, bo_ref,
               pres_out, ord_out, pool_acc):
    i = pl.program_id(0)
    h1 = h1_ref[...]
    agg = agg_ref[0] + agg_ref[1]
    h = (1.0 + eps_ref[0, 0]) * h1 + agg
    h = jnp.maximum(jnp.dot(h, w1_ref[...],
                            preferred_element_type=jnp.float32) + b1_ref[...], 0.0)
    h = jnp.dot(h, w2_ref[...], preferred_element_type=jnp.float32) + b2_ref[...]
    h2 = jnp.maximum(h, 0.0)

    bids = batch_ref[0, 0, :]
    oneh = (lax.broadcasted_iota(jnp.int32, (G, 1), 0)
            == bids[None, :]).astype(jnp.float32)
    part = jax.lax.dot_general(oneh, h2, (((1,), (0,)), ((), ())),
                               preferred_element_type=jnp.float32)

    @pl.when(i == 0)
    def _():
        pool_acc[...] = jnp.zeros_like(pool_acc)

    pool_acc[...] += part

    @pl.when(i == pl.num_programs(0) - 1)
    def _():
        pool = pool_acc[...]
        pr = jnp.maximum(jnp.dot(pool, wp1_ref[...],
                                 preferred_element_type=jnp.float32) + bp1_ref[...], 0.0)
        pres_out[...] = jax.nn.sigmoid(
            jnp.dot(pr, wp2_ref[...], preferred_element_type=jnp.float32) + bp2_ref[...])
        ord_out[...] = jnp.dot(pool, wo_ref[...],
                               preferred_element_type=jnp.float32) + bo_ref[...]


def _full(shape):
    return pl.BlockSpec(shape, lambda i: tuple(0 for _ in shape))


def kernel(x, edge_index, batch, eps1, W1a, b1a, W2a, b2a,
           eps2, W1b, b1b, W2b, b2b, Wp1, bp1, Wp2, bp2, Wo, bo):
    f32 = jnp.float32
    pad_e = E_PAD - E
    src2d = jnp.concatenate(
        [edge_index[0], jnp.zeros((pad_e,), jnp.int32)]).reshape(-1, LB)
    dst2d = jnp.concatenate(
        [edge_index[1], jnp.full((pad_e,), N, jnp.int32)]).reshape(-1, LB)
    x_pad = jnp.pad(x, ((0, 0), (0, 16 - x.shape[1])))
    W1a_p = jnp.pad(W1a, ((0, 16 - W1a.shape[0]), (0, 0)))
    batch3 = batch.reshape(NGRID, 1, BN)

    agg1 = _make_agg(1)(src2d, dst2d, x_pad)

    eps1s = eps1.reshape(1, 1)
    eps2s = eps2.reshape(1, 1)

    smem_spec = pl.BlockSpec(memory_space=pltpu.SMEM)

    h1 = pl.pallas_call(
        _mlp1_body,
        grid=(NGRID,),
        in_specs=[smem_spec,
                  pl.BlockSpec((BN, 16), lambda i: (i, 0)),
                  pl.BlockSpec((NC, BN, 16), lambda i: (0, i, 0)),
                  _full((16, 64)), _full((1, 64)), _full((64, 64)), _full((1, 64))],
        out_specs=pl.BlockSpec((BN, 64), lambda i: (i, 0)),
        out_shape=jax.ShapeDtypeStruct((N, 64), f32),
    )(eps1s, x_pad, agg1, W1a_p, b1a.reshape(1, 64), W2a, b2a.reshape(1, 64))

    h1_4 = h1.reshape(4 * N, 16)
    agg2 = _make_agg(4)(src2d, dst2d, h1_4)

    pres, ord24 = pl.pallas_call(
        _mlp2_body,
        grid=(NGRID,),
        in_specs=[smem_spec,
                  pl.BlockSpec((BN, 64), lambda i: (i, 0)),
                  pl.BlockSpec((NC, BN, 64), lambda i: (0, i, 0)),
                  pl.BlockSpec((1, 1, BN), lambda i: (i, 0, 0)),
                  _full((64, 64)), _full((1, 64)), _full((64, 64)), _full((1, 64)),
                  _full((64, 32)), _full((1, 32)), _full((32, 8)), _full((1, 8)),
                  _full((64, 24)), _full((1, 24))],
        out_specs=[_full((G, 8)), _full((G, 24))],
        out_shape=[jax.ShapeDtypeStruct((G, 8), f32),
                   jax.ShapeDtypeStruct((G, 24), f32)],
        scratch_shapes=[pltpu.VMEM((G, 64), f32)],
    )(eps2s, h1, agg2, batch3,
      W1b, b1b.reshape(1, 64), W2b, b2b.reshape(1, 64),
      Wp1, bp1.reshape(1, 32), Wp2, bp2.reshape(1, 8),
      Wo, bo.reshape(1, 24))

    return pres, ord24.reshape(G, 3, 8)


# speedup vs baseline: 11.2385x; 1.0009x over previous
import functools

import jax
import jax.numpy as jnp
from jax import lax
from jax.experimental import pallas as pl
from jax.experimental.pallas import tpu as pltpu
from jax.experimental.pallas import tpu_sc as plsc

N = 100000
E = 1600000
G = 64

NC = 2
NS = 16
LB = 128
SB = 1024
PT = 50176
E_PAD = NC * NS * PT
NBLK = PT // SB
NSUP = 7
ROWS_PER_TILE = 6256
ACC_ROWS = NS * ROWS_PER_TILE

BN = 5000
NGRID = N // BN


def _ceil_chunks(total, step):
    out, o = [], 0
    while o < total:
        out.append((o, min(step, total - o)))
        o += step
    return out


def _make_agg(num_passes):
    mesh = plsc.VectorSubcoreMesh(
        core_axis_name="c", subcore_axis_name="s", num_cores=NC, num_subcores=NS)

    @functools.partial(
        pl.kernel,
        out_type=jax.ShapeDtypeStruct((NC, ACC_ROWS, 16 * num_passes),
                                      jnp.float32),
        mesh=mesh,
        scratch_types=[
            pltpu.VMEM((56, LB), jnp.int32),
            pltpu.VMEM((56, LB), jnp.int32),
            pltpu.VMEM((SB, 16), jnp.float32),
            pltpu.VMEM_SHARED((ACC_ROWS, 16), jnp.float32),
            pltpu.SemaphoreType.DMA,
            pltpu.SemaphoreType.DMA,
        ],
        compiler_params=pltpu.CompilerParams(use_tc_tiling_on_sc=False),
    )
    def agg(src_hbm, dst_hbm, tbl, out, src_idx, dst_idx, rows, acc, sem, ssem):
        cid = lax.axis_index("c")
        sid = lax.axis_index("s")
        roff = (cid * (NS * PT) + sid * PT) // LB
        base = pl.multiple_of(sid * ROWS_PER_TILE, 8)

        for p in range(num_passes):
            def zb(r, carry):
                rows[r, :] = jnp.zeros((16,), jnp.float32)
                return carry
            lax.fori_loop(0, SB, zb, 0)
            for (o, sz) in _ceil_chunks(ROWS_PER_TILE, SB):
                pltpu.sync_copy(rows.at[pl.ds(0, sz), :],
                                acc.at[pl.ds(pl.multiple_of(base + o, 8), sz), :])
            plsc.subcore_barrier()

            def body(sc, carry):
                r0 = pl.multiple_of(roff + sc * (8 * NSUP), 8)
                pltpu.sync_copy(src_hbm.at[p, pl.ds(r0, 8 * NSUP), :], src_idx)
                pltpu.sync_copy(dst_hbm.at[pl.ds(r0, 8 * NSUP), :], dst_idx)
                for b in range(NSUP):
                    cps = [pltpu.async_copy(tbl.at[src_idx.at[b * 8 + j]],
                                            rows.at[pl.ds(j * LB, LB), :], sem)
                           for j in range(8)]
                    scs = []
                    for j in range(8):
                        cps[j].wait()
                        scs.append(
                            pltpu.async_copy(rows.at[pl.ds(j * LB, LB), :],
                                             acc.at[dst_idx.at[b * 8 + j]],
                                             ssem, add=True))
                    for s in scs:
                        s.wait()
                return carry
            lax.fori_loop(0, NBLK // NSUP, body, 0)
            plsc.subcore_barrier()

            for (o, sz) in _ceil_chunks(ROWS_PER_TILE, SB):
                o8 = pl.multiple_of(base + o, 8)
                pltpu.sync_copy(acc.at[pl.ds(o8, sz), :],
                                out.at[cid, pl.ds(o8, sz), pl.ds(16 * p, 16)])
            plsc.subcore_barrier()

    return agg


def _mlp1_body(eps_ref, xp_ref, agg_ref, w1_ref, b1_ref, w2_ref, b2_ref, out):
    x = xp_ref[...]
    agg = agg_ref[0] + agg_ref[1]
    h = (1.0 + eps_ref[0, 0]) * x + agg
    h = jnp.maximum(jnp.dot(h, w1_ref[...],
                            preferred_element_type=jnp.float32) + b1_ref[...], 0.0)
    h = jnp.dot(h, w2_ref[...], preferred_element_type=jnp.float32) + b2_ref[...]
    out[...] = jnp.maximum(h, 0.0)


def _mlp2_body(eps_ref, h1_ref, agg_ref, batch_ref,
               w1_ref, b1_ref, w2_ref, b2_ref, wp1_ref, bp1_ref,
               wp2_ref, bp2_ref, wo_ref, bo_ref,
               pres_out, ord_out, pool_acc):
    i = pl.program_id(0)
    h1 = h1_ref[...]
    agg = agg_ref[0] + agg_ref[1]
    h = (1.0 + eps_ref[0, 0]) * h1 + agg
    h = jnp.maximum(jnp.dot(h, w1_ref[...],
                            preferred_element_type=jnp.float32) + b1_ref[...], 0.0)
    h = jnp.dot(h, w2_ref[...], preferred_element_type=jnp.float32) + b2_ref[...]
    h2 = jnp.maximum(h, 0.0)

    bids = batch_ref[0, 0, :]
    oneh = (lax.broadcasted_iota(jnp.int32, (G, 1), 0)
            == bids[None, :]).astype(jnp.float32)
    part = jax.lax.dot_general(oneh, h2, (((1,), (0,)), ((), ())),
                               preferred_element_type=jnp.float32)

    @pl.when(i == 0)
    def _():
        pool_acc[...] = jnp.zeros_like(pool_acc)

    pool_acc[...] += part

    @pl.when(i == pl.num_programs(0) - 1)
    def _():
        pool = pool_acc[...]
        pr = jnp.maximum(jnp.dot(pool, wp1_ref[...],
                                 preferred_element_type=jnp.float32) + bp1_ref[...], 0.0)
        pres_out[...] = jax.nn.sigmoid(
            jnp.dot(pr, wp2_ref[...], preferred_element_type=jnp.float32) + bp2_ref[...])
        ord_out[...] = jnp.dot(pool, wo_ref[...],
                               preferred_element_type=jnp.float32) + bo_ref[...]


def _full(shape):
    return pl.BlockSpec(shape, lambda i: tuple(0 for _ in shape))


def kernel(x, edge_index, batch, eps1, W1a, b1a, W2a, b2a,
           eps2, W1b, b1b, W2b, b2b, Wp1, bp1, Wp2, bp2, Wo, bo):
    f32 = jnp.float32
    pad_e = E_PAD - E
    src2d = jnp.concatenate(
        [edge_index[0], jnp.zeros((pad_e,), jnp.int32)]).reshape(-1, LB)
    dst2d = jnp.concatenate(
        [edge_index[1], jnp.full((pad_e,), N, jnp.int32)]).reshape(-1, LB)
    x_pad = jnp.pad(x, ((0, 0), (0, 16 - x.shape[1])))
    W1a_p = jnp.pad(W1a, ((0, 16 - W1a.shape[0]), (0, 0)))
    batch3 = batch.reshape(NGRID, 1, BN)

    agg1 = _make_agg(1)(src2d[None], dst2d, x_pad)

    eps1s = eps1.reshape(1, 1)
    eps2s = eps2.reshape(1, 1)

    smem_spec = pl.BlockSpec(memory_space=pltpu.SMEM)

    h1 = pl.pallas_call(
        _mlp1_body,
        grid=(NGRID,),
        in_specs=[smem_spec,
                  pl.BlockSpec((BN, 16), lambda i: (i, 0)),
                  pl.BlockSpec((NC, BN, 16), lambda i: (0, i, 0)),
                  _full((16, 64)), _full((1, 64)), _full((64, 64)), _full((1, 64))],
        out_specs=pl.BlockSpec((BN, 64), lambda i: (i, 0)),
        out_shape=jax.ShapeDtypeStruct((N, 64), f32),
    )(eps1s, x_pad, agg1, W1a_p, b1a.reshape(1, 64), W2a, b2a.reshape(1, 64))

    h1_4 = h1.reshape(4 * N, 16)
    src4 = src2d * 4
    srcs4 = jnp.stack([src4 + q for q in range(4)])
    agg2 = _make_agg(4)(srcs4, dst2d, h1_4)

    pres, ord24 = pl.pallas_call(
        _mlp2_body,
        grid=(NGRID,),
        in_specs=[smem_spec,
                  pl.BlockSpec((BN, 64), lambda i: (i, 0)),
                  pl.BlockSpec((NC, BN, 64), lambda i: (0, i, 0)),
                  pl.BlockSpec((1, 1, BN), lambda i: (i, 0, 0)),
                  _full((64, 64)), _full((1, 64)), _full((64, 64)), _full((1, 64)),
                  _full((64, 32)), _full((1, 32)), _full((32, 8)), _full((1, 8)),
                  _full((64, 24)), _full((1, 24))],
        out_specs=[_full((G, 8)), _full((G, 24))],
        out_shape=[jax.ShapeDtypeStruct((G, 8), f32),
                   jax.ShapeDtypeStruct((G, 24), f32)],
        scratch_shapes=[pltpu.VMEM((G, 64), f32)],
    )(eps2s, h1, agg2, batch3,
      W1b, b1b.reshape(1, 64), W2b, b2b.reshape(1, 64),
      Wp1, bp1.reshape(1, 32), Wp2, bp2.reshape(1, 8),
      Wo, bo.reshape(1, 24))

    return pres, ord24.reshape(G, 3, 8)


# speedup vs baseline: 11.5189x; 1.0249x over previous
import functools

import jax
import jax.numpy as jnp
from jax import lax
from jax.experimental import pallas as pl
from jax.experimental.pallas import tpu as pltpu
from jax.experimental.pallas import tpu_sc as plsc

N = 100000
E = 1600000
G = 64

NC = 2
NS = 16
LB = 128
SB = 1024
PT = 50176
E_PAD = NC * NS * PT
NBLK = PT // SB
NSUP = 7
ROWS_PER_TILE = 6256
ACC_ROWS = NS * ROWS_PER_TILE

BN = 5000
NGRID = N // BN


def _ceil_chunks(total, step):
    out, o = [], 0
    while o < total:
        out.append((o, min(step, total - o)))
        o += step
    return out


def _make_agg(num_passes):
    mesh = plsc.VectorSubcoreMesh(
        core_axis_name="c", subcore_axis_name="s", num_cores=NC, num_subcores=NS)

    @functools.partial(
        pl.kernel,
        out_type=jax.ShapeDtypeStruct((NC, ACC_ROWS, 16 * num_passes),
                                      jnp.float32),
        mesh=mesh,
        scratch_types=[
            pltpu.VMEM((56, LB), jnp.int32),
            pltpu.VMEM((56, LB), jnp.int32),
            pltpu.VMEM((SB, 16), jnp.float32),
            pltpu.VMEM_SHARED((ACC_ROWS, 16), jnp.float32),
            pltpu.SemaphoreType.DMA,
            pltpu.SemaphoreType.DMA,
        ],
        compiler_params=pltpu.CompilerParams(use_tc_tiling_on_sc=False),
    )
    def agg(src_hbm, dst_hbm, tbl, out, src_idx, dst_idx, rows, acc, sem, ssem):
        cid = lax.axis_index("c")
        sid = lax.axis_index("s")
        roff = (cid * (NS * PT) + sid * PT) // LB
        base = pl.multiple_of(sid * ROWS_PER_TILE, 8)

        for p in range(num_passes):
            def zb(r, carry):
                rows[r, :] = jnp.zeros((16,), jnp.float32)
                return carry
            lax.fori_loop(0, SB, zb, 0)
            for (o, sz) in _ceil_chunks(ROWS_PER_TILE, SB):
                pltpu.sync_copy(rows.at[pl.ds(0, sz), :],
                                acc.at[pl.ds(pl.multiple_of(base + o, 8), sz), :])
            plsc.subcore_barrier()

            def body(sc, carry):
                r0 = pl.multiple_of(roff + sc * (8 * NSUP), 8)
                pltpu.sync_copy(src_hbm.at[p, pl.ds(r0, 8 * NSUP), :], src_idx)
                pltpu.sync_copy(dst_hbm.at[pl.ds(r0, 8 * NSUP), :], dst_idx)
                prev = None
                for b in range(NSUP):
                    cps = []
                    for j in range(8):
                        if prev is not None:
                            prev[j].wait()
                        cps.append(
                            pltpu.async_copy(tbl.at[src_idx.at[b * 8 + j]],
                                             rows.at[pl.ds(j * LB, LB), :],
                                             sem))
                    scs = []
                    for j in range(8):
                        cps[j].wait()
                        scs.append(
                            pltpu.async_copy(rows.at[pl.ds(j * LB, LB), :],
                                             acc.at[dst_idx.at[b * 8 + j]],
                                             ssem, add=True))
                    prev = scs
                for s in prev:
                    s.wait()
                return carry
            lax.fori_loop(0, NBLK // NSUP, body, 0)
            plsc.subcore_barrier()

            for (o, sz) in _ceil_chunks(ROWS_PER_TILE, SB):
                o8 = pl.multiple_of(base + o, 8)
                pltpu.sync_copy(acc.at[pl.ds(o8, sz), :],
                                out.at[cid, pl.ds(o8, sz), pl.ds(16 * p, 16)])
            plsc.subcore_barrier()

    return agg


def _mlp1_body(eps_ref, xp_ref, agg_ref, w1_ref, b1_ref, w2_ref, b2_ref, out):
    x = xp_ref[...]
    agg = agg_ref[0] + agg_ref[1]
    h = (1.0 + eps_ref[0, 0]) * x + agg
    h = jnp.maximum(jnp.dot(h, w1_ref[...],
                            preferred_element_type=jnp.float32) + b1_ref[...], 0.0)
    h = jnp.dot(h, w2_ref[...], preferred_element_type=jnp.float32) + b2_ref[...]
    out[...] = jnp.maximum(h, 0.0)


def _mlp2_body(eps_ref, h1_ref, agg_ref, batch_ref,
               w1_ref, b1_ref, w2_ref, b2_ref, wp1_ref, bp1_ref,
               wp2_ref, bp2_ref, wo_ref, bo_ref,
               pres_out, ord_out, pool_acc):
    i = pl.program_id(0)
    h1 = h1_ref[...]
    agg = agg_ref[0] + agg_ref[1]
    h = (1.0 + eps_ref[0, 0]) * h1 + agg
    h = jnp.maximum(jnp.dot(h, w1_ref[...],
                            preferred_element_type=jnp.float32) + b1_ref[...], 0.0)
    h = jnp.dot(h, w2_ref[...], preferred_element_type=jnp.float32) + b2_ref[...]
    h2 = jnp.maximum(h, 0.0)

    bids = batch_ref[0, 0, :]
    oneh = (lax.broadcasted_iota(jnp.int32, (G, 1), 0)
            == bids[None, :]).astype(jnp.float32)
    part = jax.lax.dot_general(oneh, h2, (((1,), (0,)), ((), ())),
                               preferred_element_type=jnp.float32)

    @pl.when(i == 0)
    def _():
        pool_acc[...] = jnp.zeros_like(pool_acc)

    pool_acc[...] += part

    @pl.when(i == pl.num_programs(0) - 1)
    def _():
        pool = pool_acc[...]
        pr = jnp.maximum(jnp.dot(pool, wp1_ref[...],
                                 preferred_element_type=jnp.float32) + bp1_ref[...], 0.0)
        pres_out[...] = jax.nn.sigmoid(
            jnp.dot(pr, wp2_ref[...], preferred_element_type=jnp.float32) + bp2_ref[...])
        ord_out[...] = jnp.dot(pool, wo_ref[...],
                               preferred_element_type=jnp.float32) + bo_ref[...]


def _full(shape):
    return pl.BlockSpec(shape, lambda i: tuple(0 for _ in shape))


def kernel(x, edge_index, batch, eps1, W1a, b1a, W2a, b2a,
           eps2, W1b, b1b, W2b, b2b, Wp1, bp1, Wp2, bp2, Wo, bo):
    f32 = jnp.float32
    pad_e = E_PAD - E
    src2d = jnp.concatenate(
        [edge_index[0], jnp.zeros((pad_e,), jnp.int32)]).reshape(-1, LB)
    dst2d = jnp.concatenate(
        [edge_index[1], jnp.full((pad_e,), N, jnp.int32)]).reshape(-1, LB)
    x_pad = jnp.pad(x, ((0, 0), (0, 16 - x.shape[1])))
    W1a_p = jnp.pad(W1a, ((0, 16 - W1a.shape[0]), (0, 0)))
    batch3 = batch.reshape(NGRID, 1, BN)

    agg1 = _make_agg(1)(src2d[None], dst2d, x_pad)

    eps1s = eps1.reshape(1, 1)
    eps2s = eps2.reshape(1, 1)

    smem_spec = pl.BlockSpec(memory_space=pltpu.SMEM)

    h1 = pl.pallas_call(
        _mlp1_body,
        grid=(NGRID,),
        in_specs=[smem_spec,
                  pl.BlockSpec((BN, 16), lambda i: (i, 0)),
                  pl.BlockSpec((NC, BN, 16), lambda i: (0, i, 0)),
                  _full((16, 64)), _full((1, 64)), _full((64, 64)), _full((1, 64))],
        out_specs=pl.BlockSpec((BN, 64), lambda i: (i, 0)),
        out_shape=jax.ShapeDtypeStruct((N, 64), f32),
    )(eps1s, x_pad, agg1, W1a_p, b1a.reshape(1, 64), W2a, b2a.reshape(1, 64))

    h1_4 = h1.reshape(4 * N, 16)
    src4 = src2d * 4
    srcs4 = jnp.stack([src4 + q for q in range(4)])
    agg2 = _make_agg(4)(srcs4, dst2d, h1_4)

    pres, ord24 = pl.pallas_call(
        _mlp2_body,
        grid=(NGRID,),
        in_specs=[smem_spec,
                  pl.BlockSpec((BN, 64), lambda i: (i, 0)),
                  pl.BlockSpec((NC, BN, 64), lambda i: (0, i, 0)),
                  pl.BlockSpec((1, 1, BN), lambda i: (i, 0, 0)),
                  _full((64, 64)), _full((1, 64)), _full((64, 64)), _full((1, 64)),
                  _full((64, 32)), _full((1, 32)), _full((32, 8)), _full((1, 8)),
                  _full((64, 24)), _full((1, 24))],
        out_specs=[_full((G, 8)), _full((G, 24))],
        out_shape=[jax.ShapeDtypeStruct((G, 8), f32),
                   jax.ShapeDtypeStruct((G, 24), f32)],
        scratch_shapes=[pltpu.VMEM((G, 64), f32)],
    )(eps2s, h1, agg2, batch3,
      W1b, b1b.reshape(1, 64), W2b, b2b.reshape(1, 64),
      Wp1, bp1.reshape(1, 32), Wp2, bp2.reshape(1, 8),
      Wo, bo.reshape(1, 24))

    return pres, ord24.reshape(G, 3, 8)


# speedup vs baseline: 12.3880x; 1.0755x over previous
import functools

import jax
import jax.numpy as jnp
from jax import lax
from jax.experimental import pallas as pl
from jax.experimental.pallas import tpu as pltpu
from jax.experimental.pallas import tpu_sc as plsc

N = 100000
E = 1600000
G = 64

NC = 2
NS = 16
LB = 128
SB = 1024
PT = 50176
E_PAD = NC * NS * PT
NBLK = PT // SB
NSUP = 7
ROWS_PER_TILE = 6256
ACC_ROWS = NS * ROWS_PER_TILE

BN = 5000
NGRID = N // BN


def _ceil_chunks(total, step):
    out, o = [], 0
    while o < total:
        out.append((o, min(step, total - o)))
        o += step
    return out


def _make_agg(num_passes):
    mesh = plsc.VectorSubcoreMesh(
        core_axis_name="c", subcore_axis_name="s", num_cores=NC, num_subcores=NS)

    @functools.partial(
        pl.kernel,
        out_type=jax.ShapeDtypeStruct((NC, ACC_ROWS, 16 * num_passes),
                                      jnp.float32),
        mesh=mesh,
        scratch_types=[
            pltpu.VMEM((56, LB), jnp.int32),
            pltpu.VMEM((56, LB), jnp.int32),
            pltpu.VMEM((SB, 16), jnp.float32),
            pltpu.VMEM_SHARED((ACC_ROWS, 16), jnp.float32),
            pltpu.SemaphoreType.DMA,
            pltpu.SemaphoreType.DMA,
        ],
        compiler_params=pltpu.CompilerParams(use_tc_tiling_on_sc=False),
    )
    def agg(src_hbm, dst_hbm, tbl, out, src_idx, dst_idx, rows, acc, sem, ssem):
        cid = lax.axis_index("c")
        sid = lax.axis_index("s")
        roff = (cid * (NS * PT) + sid * PT) // LB
        base = pl.multiple_of(sid * ROWS_PER_TILE, 8)

        for p in range(num_passes):
            def zb(r, carry):
                rows[r, :] = jnp.zeros((16,), jnp.float32)
                return carry
            lax.fori_loop(0, SB, zb, 0)
            for (o, sz) in _ceil_chunks(ROWS_PER_TILE, SB):
                pltpu.sync_copy(rows.at[pl.ds(0, sz), :],
                                acc.at[pl.ds(pl.multiple_of(base + o, 8), sz), :])
            plsc.subcore_barrier()

            def body(sc, carry):
                r0 = pl.multiple_of(roff + sc * (8 * NSUP), 8)
                pltpu.sync_copy(src_hbm.at[p, pl.ds(r0, 8 * NSUP), :], src_idx)
                pltpu.sync_copy(dst_hbm.at[pl.ds(r0, 8 * NSUP), :], dst_idx)
                prev = None
                for b in range(NSUP):
                    cps = []
                    for j in range(8):
                        if prev is not None:
                            prev[j].wait()
                        cps.append(
                            pltpu.async_copy(tbl.at[src_idx.at[b * 8 + j]],
                                             rows.at[pl.ds(j * LB, LB), :],
                                             sem))
                    scs = []
                    for j in range(8):
                        cps[j].wait()
                        scs.append(
                            pltpu.async_copy(rows.at[pl.ds(j * LB, LB), :],
                                             acc.at[dst_idx.at[b * 8 + j]],
                                             ssem, add=True))
                    prev = scs
                for s in prev:
                    s.wait()
                return carry
            lax.fori_loop(0, NBLK // NSUP, body, 0)
            plsc.subcore_barrier()

            for (o, sz) in _ceil_chunks(ROWS_PER_TILE, SB):
                o8 = pl.multiple_of(base + o, 8)
                pltpu.sync_copy(acc.at[pl.ds(o8, sz), :],
                                out.at[cid, pl.ds(o8, sz), pl.ds(16 * p, 16)])
            plsc.subcore_barrier()

    return agg


PT2 = E_PAD // NS
NBLK2 = PT2 // SB
NBI = E_PAD // LB


def _make_agg2():
    mesh = plsc.VectorSubcoreMesh(
        core_axis_name="c", subcore_axis_name="s", num_cores=NC, num_subcores=NS)

    @functools.partial(
        pl.kernel,
        out_type=jax.ShapeDtypeStruct((ACC_ROWS, 64), jnp.float32),
        mesh=mesh,
        scratch_types=[
            pltpu.VMEM((56, LB), jnp.int32),
            pltpu.VMEM((56, LB), jnp.int32),
            pltpu.VMEM((SB, 16), jnp.float32),
            pltpu.VMEM_SHARED((ACC_ROWS, 16), jnp.float32),
            pltpu.SemaphoreType.DMA,
            pltpu.SemaphoreType.DMA,
        ],
        compiler_params=pltpu.CompilerParams(use_tc_tiling_on_sc=False),
    )
    def agg(src_hbm, dst_hbm, tbl, out, src_idx, dst_idx, rows, acc, sem, ssem):
        cid = lax.axis_index("c")
        sid = lax.axis_index("s")
        roff = (sid * PT2) // LB
        base = pl.multiple_of(sid * ROWS_PER_TILE, 8)

        for p in range(2):
            q = cid * 2 + p

            def zb(r, carry):
                rows[r, :] = jnp.zeros((16,), jnp.float32)
                return carry
            lax.fori_loop(0, SB, zb, 0)
            for (o, sz) in _ceil_chunks(ROWS_PER_TILE, SB):
                pltpu.sync_copy(rows.at[pl.ds(0, sz), :],
                                acc.at[pl.ds(pl.multiple_of(base + o, 8), sz), :])
            plsc.subcore_barrier()

            def body(sc, carry):
                r0 = pl.multiple_of(q * NBI + roff + sc * (8 * NSUP), 8)
                pltpu.sync_copy(src_hbm.at[pl.ds(r0, 8 * NSUP), :], src_idx)
                d0 = pl.multiple_of(roff + sc * (8 * NSUP), 8)
                pltpu.sync_copy(dst_hbm.at[pl.ds(d0, 8 * NSUP), :], dst_idx)
                prev = None
                for b in range(NSUP):
                    cps = []
                    for j in range(8):
                        if prev is not None:
                            prev[j].wait()
                        cps.append(
                            pltpu.async_copy(tbl.at[src_idx.at[b * 8 + j]],
                                             rows.at[pl.ds(j * LB, LB), :],
                                             sem))
                    scs = []
                    for j in range(8):
                        cps[j].wait()
                        scs.append(
                            pltpu.async_copy(rows.at[pl.ds(j * LB, LB), :],
                                             acc.at[dst_idx.at[b * 8 + j]],
                                             ssem, add=True))
                    prev = scs
                for s in prev:
                    s.wait()
                return carry
            lax.fori_loop(0, NBLK2 // NSUP, body, 0)
            plsc.subcore_barrier()

            coff = pl.multiple_of(q * 16, 16)
            for (o, sz) in _ceil_chunks(ROWS_PER_TILE, SB):
                o8 = pl.multiple_of(base + o, 8)
                pltpu.sync_copy(acc.at[pl.ds(o8, sz), :],
                                out.at[pl.ds(o8, sz), pl.ds(coff, 16)])
            plsc.subcore_barrier()

    return agg


def _mlp1_body(eps_ref, xp_ref, agg_ref, w1_ref, b1_ref, w2_ref, b2_ref, out):
    x = xp_ref[...]
    agg = agg_ref[0] + agg_ref[1]
    h = (1.0 + eps_ref[0, 0]) * x + agg
    h = jnp.maximum(jnp.dot(h, w1_ref[...],
                            preferred_element_type=jnp.float32) + b1_ref[...], 0.0)
    h = jnp.dot(h, w2_ref[...], preferred_element_type=jnp.float32) + b2_ref[...]
    out[...] = jnp.maximum(h, 0.0)


def _mlp2_body(eps_ref, h1_ref, agg_ref, batch_ref,
               w1_ref, b1_ref, w2_ref, b2_ref, wp1_ref, bp1_ref,
               wp2_ref, bp2_ref, wo_ref, bo_ref,
               pres_out, ord_out, pool_acc):
    i = pl.program_id(0)
    h1 = h1_ref[...]
    h = (1.0 + eps_ref[0, 0]) * h1 + agg_ref[...]
    h = jnp.maximum(jnp.dot(h, w1_ref[...],
                            preferred_element_type=jnp.float32) + b1_ref[...], 0.0)
    h = jnp.dot(h, w2_ref[...], preferred_element_type=jnp.float32) + b2_ref[...]
    h2 = jnp.maximum(h, 0.0)

    bids = batch_ref[0, 0, :]
    oneh = (lax.broadcasted_iota(jnp.int32, (G, 1), 0)
            == bids[None, :]).astype(jnp.float32)
    part = jax.lax.dot_general(oneh, h2, (((1,), (0,)), ((), ())),
                               preferred_element_type=jnp.float32)

    @pl.when(i == 0)
    def _():
        pool_acc[...] = jnp.zeros_like(pool_acc)

    pool_acc[...] += part

    @pl.when(i == pl.num_programs(0) - 1)
    def _():
        pool = pool_acc[...]
        pr = jnp.maximum(jnp.dot(pool, wp1_ref[...],
                                 preferred_element_type=jnp.float32) + bp1_ref[...], 0.0)
        pres_out[...] = jax.nn.sigmoid(
            jnp.dot(pr, wp2_ref[...], preferred_element_type=jnp.float32) + bp2_ref[...])
        ord_out[...] = jnp.dot(pool, wo_ref[...],
                               preferred_element_type=jnp.float32) + bo_ref[...]


def _full(shape):
    return pl.BlockSpec(shape, lambda i: tuple(0 for _ in shape))


def kernel(x, edge_index, batch, eps1, W1a, b1a, W2a, b2a,
           eps2, W1b, b1b, W2b, b2b, Wp1, bp1, Wp2, bp2, Wo, bo):
    f32 = jnp.float32
    pad_e = E_PAD - E
    src2d = jnp.concatenate(
        [edge_index[0], jnp.zeros((pad_e,), jnp.int32)]).reshape(-1, LB)
    dst2d = jnp.concatenate(
        [edge_index[1], jnp.full((pad_e,), N, jnp.int32)]).reshape(-1, LB)
    x_pad = jnp.pad(x, ((0, 0), (0, 16 - x.shape[1])))
    W1a_p = jnp.pad(W1a, ((0, 16 - W1a.shape[0]), (0, 0)))
    batch3 = batch.reshape(NGRID, 1, BN)

    agg1 = _make_agg(1)(src2d[None], dst2d, x_pad)

    eps1s = eps1.reshape(1, 1)
    eps2s = eps2.reshape(1, 1)

    smem_spec = pl.BlockSpec(memory_space=pltpu.SMEM)

    h1 = pl.pallas_call(
        _mlp1_body,
        grid=(NGRID,),
        in_specs=[smem_spec,
                  pl.BlockSpec((BN, 16), lambda i: (i, 0)),
                  pl.BlockSpec((NC, BN, 16), lambda i: (0, i, 0)),
                  _full((16, 64)), _full((1, 64)), _full((64, 64)), _full((1, 64))],
        out_specs=pl.BlockSpec((BN, 64), lambda i: (i, 0)),
        out_shape=jax.ShapeDtypeStruct((N, 64), f32),
    )(eps1s, x_pad, agg1, W1a_p, b1a.reshape(1, 64), W2a, b2a.reshape(1, 64))

    h1_4 = h1.reshape(4 * N, 16)
    src4 = src2d * 4
    srcs4 = jnp.concatenate([src4 + q for q in range(4)])
    agg2 = _make_agg2()(srcs4, dst2d, h1_4)

    pres, ord24 = pl.pallas_call(
        _mlp2_body,
        grid=(NGRID,),
        in_specs=[smem_spec,
                  pl.BlockSpec((BN, 64), lambda i: (i, 0)),
                  pl.BlockSpec((BN, 64), lambda i: (i, 0)),
                  pl.BlockSpec((1, 1, BN), lambda i: (i, 0, 0)),
                  _full((64, 64)), _full((1, 64)), _full((64, 64)), _full((1, 64)),
                  _full((64, 32)), _full((1, 32)), _full((32, 8)), _full((1, 8)),
                  _full((64, 24)), _full((1, 24))],
        out_specs=[_full((G, 8)), _full((G, 24))],
        out_shape=[jax.ShapeDtypeStruct((G, 8), f32),
                   jax.ShapeDtypeStruct((G, 24), f32)],
        scratch_shapes=[pltpu.VMEM((G, 64), f32)],
    )(eps2s, h1, agg2, batch3,
      W1b, b1b.reshape(1, 64), W2b, b2b.reshape(1, 64),
      Wp1, bp1.reshape(1, 32), Wp2, bp2.reshape(1, 8),
      Wo, bo.reshape(1, 24))

    return pres, ord24.reshape(G, 3, 8)


# speedup vs baseline: 12.4650x; 1.0062x over previous
import functools

import jax
import jax.numpy as jnp
from jax import lax
from jax.experimental import pallas as pl
from jax.experimental.pallas import tpu as pltpu
from jax.experimental.pallas import tpu_sc as plsc

N = 100000
E = 1600000
G = 64

NC = 2
NS = 16
LB = 128
SB = 1024
PT = 50176
E_PAD = NC * NS * PT
NBLK = PT // SB
NSUP = 7
ROWS_PER_TILE = 6256
ACC_ROWS = NS * ROWS_PER_TILE

BN = 5000
NGRID = N // BN


def _ceil_chunks(total, step):
    out, o = [], 0
    while o < total:
        out.append((o, min(step, total - o)))
        o += step
    return out


def _make_agg(num_passes):
    mesh = plsc.VectorSubcoreMesh(
        core_axis_name="c", subcore_axis_name="s", num_cores=NC, num_subcores=NS)

    @functools.partial(
        pl.kernel,
        out_type=jax.ShapeDtypeStruct((NC, ACC_ROWS, 16 * num_passes),
                                      jnp.float32),
        mesh=mesh,
        scratch_types=[
            pltpu.VMEM((56, LB), jnp.int32),
            pltpu.VMEM((56, LB), jnp.int32),
            pltpu.VMEM((SB, 16), jnp.float32),
            pltpu.VMEM_SHARED((ACC_ROWS, 16), jnp.float32),
            pltpu.SemaphoreType.DMA,
            pltpu.SemaphoreType.DMA,
        ],
        compiler_params=pltpu.CompilerParams(use_tc_tiling_on_sc=False),
    )
    def agg(src_hbm, dst_hbm, tbl, out, src_idx, dst_idx, rows, acc, sem, ssem):
        cid = lax.axis_index("c")
        sid = lax.axis_index("s")
        roff = (cid * (NS * PT) + sid * PT) // LB
        base = pl.multiple_of(sid * ROWS_PER_TILE, 8)

        for p in range(num_passes):
            def zb(r, carry):
                rows[r, :] = jnp.zeros((16,), jnp.float32)
                return carry
            lax.fori_loop(0, SB, zb, 0)
            for (o, sz) in _ceil_chunks(ROWS_PER_TILE, SB):
                pltpu.sync_copy(rows.at[pl.ds(0, sz), :],
                                acc.at[pl.ds(pl.multiple_of(base + o, 8), sz), :])
            plsc.subcore_barrier()

            def body(sc, carry):
                r0 = pl.multiple_of(roff + sc * (8 * NSUP), 8)
                pltpu.sync_copy(src_hbm.at[p, pl.ds(r0, 8 * NSUP), :], src_idx)
                pltpu.sync_copy(dst_hbm.at[pl.ds(r0, 8 * NSUP), :], dst_idx)
                prev = None
                for b in range(NSUP):
                    cps = []
                    for j in range(8):
                        if prev is not None:
                            prev[j].wait()
                        cps.append(
                            pltpu.async_copy(tbl.at[src_idx.at[b * 8 + j]],
                                             rows.at[pl.ds(j * LB, LB), :],
                                             sem))
                    scs = []
                    for j in range(8):
                        cps[j].wait()
                        scs.append(
                            pltpu.async_copy(rows.at[pl.ds(j * LB, LB), :],
                                             acc.at[dst_idx.at[b * 8 + j]],
                                             ssem, add=True))
                    prev = scs
                for s in prev:
                    s.wait()
                return carry
            lax.fori_loop(0, NBLK // NSUP, body, 0)
            plsc.subcore_barrier()

            for (o, sz) in _ceil_chunks(ROWS_PER_TILE, SB):
                o8 = pl.multiple_of(base + o, 8)
                pltpu.sync_copy(acc.at[pl.ds(o8, sz), :],
                                out.at[cid, pl.ds(o8, sz), pl.ds(16 * p, 16)])
            plsc.subcore_barrier()

    return agg


PT2 = E_PAD // NS
NBLK2 = PT2 // SB
NBI = E_PAD // LB


def _make_agg2():
    mesh = plsc.VectorSubcoreMesh(
        core_axis_name="c", subcore_axis_name="s", num_cores=NC, num_subcores=NS)

    @functools.partial(
        pl.kernel,
        out_type=jax.ShapeDtypeStruct((ACC_ROWS, 64), jnp.float32),
        mesh=mesh,
        scratch_types=[
            pltpu.VMEM((56, LB), jnp.int32),
            pltpu.VMEM((56, LB), jnp.int32),
            pltpu.VMEM((SB, 16), jnp.float32),
            pltpu.VMEM_SHARED((ACC_ROWS, 16), jnp.float32),
            pltpu.SemaphoreType.DMA,
            pltpu.SemaphoreType.DMA,
        ],
        compiler_params=pltpu.CompilerParams(use_tc_tiling_on_sc=False),
    )
    def agg(src_hbm, dst_hbm, tbl, out, src_idx, dst_idx, rows, acc, sem, ssem):
        cid = lax.axis_index("c")
        sid = lax.axis_index("s")
        roff = (sid * PT2) // LB
        base = pl.multiple_of(sid * ROWS_PER_TILE, 8)

        for p in range(2):
            q = cid * 2 + p

            def zb(r, carry):
                rows[r, :] = jnp.zeros((16,), jnp.float32)
                return carry
            lax.fori_loop(0, SB, zb, 0)
            for (o, sz) in _ceil_chunks(ROWS_PER_TILE, SB):
                pltpu.sync_copy(rows.at[pl.ds(0, sz), :],
                                acc.at[pl.ds(pl.multiple_of(base + o, 8), sz), :])
            plsc.subcore_barrier()

            def body(sc, carry):
                d0 = pl.multiple_of(roff + sc * (8 * NSUP), 8)
                pltpu.sync_copy(src_hbm.at[pl.ds(d0, 8 * NSUP), :], src_idx)
                pltpu.sync_copy(dst_hbm.at[pl.ds(d0, 8 * NSUP), :], dst_idx)

                def fix(r, c2):
                    for c in range(8):
                        sl = pl.ds(c * 16, 16)
                        src_idx[r, sl] = src_idx[r, sl] * 4 + q
                    return c2
                lax.fori_loop(0, 8 * NSUP, fix, 0)
                prev = None
                for b in range(NSUP):
                    cps = []
                    for j in range(8):
                        if prev is not None:
                            prev[j].wait()
                        cps.append(
                            pltpu.async_copy(tbl.at[src_idx.at[b * 8 + j]],
                                             rows.at[pl.ds(j * LB, LB), :],
                                             sem))
                    scs = []
                    for j in range(8):
                        cps[j].wait()
                        scs.append(
                            pltpu.async_copy(rows.at[pl.ds(j * LB, LB), :],
                                             acc.at[dst_idx.at[b * 8 + j]],
                                             ssem, add=True))
                    prev = scs
                for s in prev:
                    s.wait()
                return carry
            lax.fori_loop(0, NBLK2 // NSUP, body, 0)
            plsc.subcore_barrier()

            coff = pl.multiple_of(q * 16, 16)
            for (o, sz) in _ceil_chunks(ROWS_PER_TILE, SB):
                o8 = pl.multiple_of(base + o, 8)
                pltpu.sync_copy(acc.at[pl.ds(o8, sz), :],
                                out.at[pl.ds(o8, sz), pl.ds(coff, 16)])
            plsc.subcore_barrier()

    return agg


def _mlp1_body(eps_ref, xp_ref, agg_ref, w1_ref, b1_ref, w2_ref, b2_ref, out):
    x = xp_ref[...]
    agg = agg_ref[0] + agg_ref[1]
    h = (1.0 + eps_ref[0, 0]) * x + agg
    h = jnp.maximum(jnp.dot(h, w1_ref[...],
                            preferred_element_type=jnp.float32) + b1_ref[...], 0.0)
    h = jnp.dot(h, w2_ref[...], preferred_element_type=jnp.float32) + b2_ref[...]
    out[...] = jnp.maximum(h, 0.0)


def _mlp2_body(eps_ref, h1_ref, agg_ref, batch_ref,
               w1_ref, b1_ref, w2_ref, b2_ref, wp1_ref, bp1_ref,
               wp2_ref, bp2_ref, wo_ref, bo_ref,
               pres_out, ord_out, pool_acc):
    i = pl.program_id(0)
    h1 = h1_ref[...]
    h = (1.0 + eps_ref[0, 0]) * h1 + agg_ref[...]
    h = jnp.maximum(jnp.dot(h, w1_ref[...],
                            preferred_element_type=jnp.float32) + b1_ref[...], 0.0)
    h = jnp.dot(h, w2_ref[...], preferred_element_type=jnp.float32) + b2_ref[...]
    h2 = jnp.maximum(h, 0.0)

    bids = batch_ref[0, 0, :]
    oneh = (lax.broadcasted_iota(jnp.int32, (G, 1), 0)
            == bids[None, :]).astype(jnp.float32)
    part = jax.lax.dot_general(oneh, h2, (((1,), (0,)), ((), ())),
                               preferred_element_type=jnp.float32)

    @pl.when(i == 0)
    def _():
        pool_acc[...] = jnp.zeros_like(pool_acc)

    pool_acc[...] += part

    @pl.when(i == pl.num_programs(0) - 1)
    def _():
        pool = pool_acc[...]
        pr = jnp.maximum(jnp.dot(pool, wp1_ref[...],
                                 preferred_element_type=jnp.float32) + bp1_ref[...], 0.0)
        pres_out[...] = jax.nn.sigmoid(
            jnp.dot(pr, wp2_ref[...], preferred_element_type=jnp.float32) + bp2_ref[...])
        ord_out[...] = jnp.dot(pool, wo_ref[...],
                               preferred_element_type=jnp.float32) + bo_ref[...]


def _full(shape):
    return pl.BlockSpec(shape, lambda i: tuple(0 for _ in shape))


def kernel(x, edge_index, batch, eps1, W1a, b1a, W2a, b2a,
           eps2, W1b, b1b, W2b, b2b, Wp1, bp1, Wp2, bp2, Wo, bo):
    f32 = jnp.float32
    pad_e = E_PAD - E
    src2d = jnp.concatenate(
        [edge_index[0], jnp.zeros((pad_e,), jnp.int32)]).reshape(-1, LB)
    dst2d = jnp.concatenate(
        [edge_index[1], jnp.full((pad_e,), N, jnp.int32)]).reshape(-1, LB)
    x_pad = jnp.pad(x, ((0, 0), (0, 16 - x.shape[1])))
    W1a_p = jnp.pad(W1a, ((0, 16 - W1a.shape[0]), (0, 0)))
    batch3 = batch.reshape(NGRID, 1, BN)

    agg1 = _make_agg(1)(src2d[None], dst2d, x_pad)

    eps1s = eps1.reshape(1, 1)
    eps2s = eps2.reshape(1, 1)

    smem_spec = pl.BlockSpec(memory_space=pltpu.SMEM)

    h1 = pl.pallas_call(
        _mlp1_body,
        grid=(NGRID,),
        in_specs=[smem_spec,
                  pl.BlockSpec((BN, 16), lambda i: (i, 0)),
                  pl.BlockSpec((NC, BN, 16), lambda i: (0, i, 0)),
                  _full((16, 64)), _full((1, 64)), _full((64, 64)), _full((1, 64))],
        out_specs=pl.BlockSpec((BN, 64), lambda i: (i, 0)),
        out_shape=jax.ShapeDtypeStruct((N, 64), f32),
    )(eps1s, x_pad, agg1, W1a_p, b1a.reshape(1, 64), W2a, b2a.reshape(1, 64))

    h1_4 = h1.reshape(4 * N, 16)
    agg2 = _make_agg2()(src2d, dst2d, h1_4)

    pres, ord24 = pl.pallas_call(
        _mlp2_body,
        grid=(NGRID,),
        in_specs=[smem_spec,
                  pl.BlockSpec((BN, 64), lambda i: (i, 0)),
                  pl.BlockSpec((BN, 64), lambda i: (i, 0)),
                  pl.BlockSpec((1, 1, BN), lambda i: (i, 0, 0)),
                  _full((64, 64)), _full((1, 64)), _full((64, 64)), _full((1, 64)),
                  _full((64, 32)), _full((1, 32)), _full((32, 8)), _full((1, 8)),
                  _full((64, 24)), _full((1, 24))],
        out_specs=[_full((G, 8)), _full((G, 24))],
        out_shape=[jax.ShapeDtypeStruct((G, 8), f32),
                   jax.ShapeDtypeStruct((G, 24), f32)],
        scratch_shapes=[pltpu.VMEM((G, 64), f32)],
    )(eps2s, h1, agg2, batch3,
      W1b, b1b.reshape(1, 64), W2b, b2b.reshape(1, 64),
      Wp1, bp1.reshape(1, 32), Wp2, bp2.reshape(1, 8),
      Wo, bo.reshape(1, 24))

    return pres, ord24.reshape(G, 3, 8)
